# baseline (device time: 15744 ns/iter reference)
import jax
import jax.numpy as jnp
from jax import lax
from jax.experimental import pallas as pl
from jax.experimental.pallas import tpu as pltpu

N_DEV = 4


def kernel(A, B):
    M, K = A.shape
    _, N = B.shape
    m_per = M // N_DEV

    def body(a_hbm, b_hbm, out_ref, a_ref, b_ref, acc_ref, recv_ref,
             load_sems, send_sems, recv_sems):
        my = lax.axis_index("i")

        b_cp = pltpu.make_async_copy(b_hbm, b_ref, load_sems.at[0])
        b_cp.start()
        a_cp = pltpu.make_async_copy(a_hbm, a_ref, load_sems.at[1])
        a_cp.start()

        barrier_sem = pltpu.get_barrier_semaphore()
        for off in (1, 2, 3):
            pl.semaphore_signal(
                barrier_sem, inc=1,
                device_id=((my + off) % N_DEV,),
                device_id_type=pl.DeviceIdType.MESH,
            )
        pl.semaphore_wait(barrier_sem, 3)

        b_cp.wait()
        a_cp.wait()
        b_bf = b_ref[...].astype(jnp.bfloat16)

        rdmas = []
        for slot, off in enumerate((2, 1, 3)):
            p = (my + off) % N_DEV
            a_chunk = a_ref[pl.ds(p * m_per, m_per), :].astype(jnp.bfloat16)
            acc_ref[slot, :, :] = jnp.dot(
                a_chunk, b_bf, preferred_element_type=jnp.float32
            ).astype(jnp.bfloat16)
            rdma = pltpu.make_async_remote_copy(
                src_ref=acc_ref.at[slot],
                dst_ref=recv_ref.at[my],
                send_sem=send_sems.at[off - 1],
                recv_sem=recv_sems.at[off - 1],
                device_id=(p,),
                device_id_type=pl.DeviceIdType.MESH,
            )
            rdma.start()
            rdmas.append(rdma)

        a_own = a_ref[pl.ds(my * m_per, m_per), :].astype(jnp.bfloat16)
        recv_ref[my, :, :] = jnp.dot(
            a_own, b_bf, preferred_element_type=jnp.float32
        ).astype(jnp.bfloat16)

        for rdma in rdmas:
            rdma.wait_recv()

        out_ref[...] = jnp.sum(
            recv_ref[...].astype(jnp.float32), axis=0
        ).astype(jnp.bfloat16)

        for rdma in rdmas:
            rdma.wait_send()

    return pl.pallas_call(
        body,
        out_shape=jax.ShapeDtypeStruct((m_per, N), jnp.bfloat16),
        in_specs=[
            pl.BlockSpec(memory_space=pl.ANY),
            pl.BlockSpec(memory_space=pl.ANY),
        ],
        out_specs=pl.BlockSpec(memory_space=pltpu.VMEM),
        scratch_shapes=[
            pltpu.VMEM((M, K), jnp.float32),
            pltpu.VMEM((K, N), jnp.float32),
            pltpu.VMEM((N_DEV - 1, m_per, N), jnp.bfloat16),
            pltpu.VMEM((N_DEV, m_per, N), jnp.bfloat16),
            pltpu.SemaphoreType.DMA((2,)),
            pltpu.SemaphoreType.DMA((N_DEV - 1,)),
            pltpu.SemaphoreType.DMA((N_DEV - 1,)),
        ],
        compiler_params=pltpu.CompilerParams(collective_id=0),
    )(A, B)


# device time: 14750 ns/iter; 1.0674x vs baseline; 1.0674x over previous
import jax
import jax.numpy as jnp
from jax import lax
from jax.experimental import pallas as pl
from jax.experimental.pallas import tpu as pltpu

N_DEV = 4


def kernel(A, B):
    M, K = A.shape
    _, N = B.shape
    m_per = M // N_DEV

    def body(a_ref, b_ref, out_ref, acc_ref, recv_ref, send_sems, recv_sems):
        my = lax.axis_index("i")

        barrier_sem = pltpu.get_barrier_semaphore()
        for off in (1, 2, 3):
            pl.semaphore_signal(
                barrier_sem, inc=1,
                device_id=((my + off) % N_DEV,),
                device_id_type=pl.DeviceIdType.MESH,
            )

        b_bf = b_ref[...].astype(jnp.bfloat16)

        for slot, off in enumerate((2, 1, 3)):
            p = (my + off) % N_DEV
            a_chunk = a_ref[pl.ds(p * m_per, m_per), :].astype(jnp.bfloat16)
            acc_ref[slot, :, :] = jnp.dot(
                a_chunk, b_bf, preferred_element_type=jnp.float32
            ).astype(jnp.bfloat16)

        pl.semaphore_wait(barrier_sem, 3)

        rdmas = []
        for slot, off in enumerate((2, 1, 3)):
            p = (my + off) % N_DEV
            rdma = pltpu.make_async_remote_copy(
                src_ref=acc_ref.at[slot],
                dst_ref=recv_ref.at[my],
                send_sem=send_sems.at[off - 1],
                recv_sem=recv_sems.at[off - 1],
                device_id=(p,),
                device_id_type=pl.DeviceIdType.MESH,
            )
            rdma.start()
            rdmas.append(rdma)

        a_own = a_ref[pl.ds(my * m_per, m_per), :].astype(jnp.bfloat16)
        recv_ref[my, :, :] = jnp.dot(
            a_own, b_bf, preferred_element_type=jnp.float32
        ).astype(jnp.bfloat16)

        for rdma in rdmas:
            rdma.wait_recv()

        out_ref[...] = jnp.sum(
            recv_ref[...].astype(jnp.float32), axis=0
        ).astype(jnp.bfloat16)

        for rdma in rdmas:
            rdma.wait_send()

    return pl.pallas_call(
        body,
        out_shape=jax.ShapeDtypeStruct((m_per, N), jnp.bfloat16),
        in_specs=[
            pl.BlockSpec(memory_space=pltpu.VMEM),
            pl.BlockSpec(memory_space=pltpu.VMEM),
        ],
        out_specs=pl.BlockSpec(memory_space=pltpu.VMEM),
        scratch_shapes=[
            pltpu.VMEM((N_DEV - 1, m_per, N), jnp.bfloat16),
            pltpu.VMEM((N_DEV, m_per, N), jnp.bfloat16),
            pltpu.SemaphoreType.DMA((N_DEV - 1,)),
            pltpu.SemaphoreType.DMA((N_DEV - 1,)),
        ],
        compiler_params=pltpu.CompilerParams(collective_id=0),
    )(A, B)


# device time: 13086 ns/iter; 1.2031x vs baseline; 1.1272x over previous
import jax
import jax.numpy as jnp
from jax import lax
from jax.experimental import pallas as pl
from jax.experimental.pallas import tpu as pltpu

N_DEV = 4


def kernel(A, B):
    M, K = A.shape
    _, N = B.shape
    m_per = M // N_DEV
    h_per = m_per // 2

    def body(a_ref, b_ref, out_ref, stage_ref, own_ref, recv_ref,
             send_sems, recv_sems):
        my = lax.axis_index("i")
        left = (my - 1) % N_DEV
        right = (my + 1) % N_DEV

        barrier_sem = pltpu.get_barrier_semaphore()
        for nbr in (left, right):
            pl.semaphore_signal(
                barrier_sem, inc=1,
                device_id=(nbr,), device_id_type=pl.DeviceIdType.MESH,
            )

        b_bf = b_ref[...].astype(jnp.bfloat16)

        def pchunk(c):
            a_c = a_ref[pl.ds(c * m_per, m_per), :].astype(jnp.bfloat16)
            return jnp.dot(
                a_c, b_bf, preferred_element_type=jnp.float32
            ).astype(jnp.bfloat16)

        def send(src_slot, dst_slot, peer, sem):
            rdma = pltpu.make_async_remote_copy(
                src_ref=stage_ref.at[src_slot],
                dst_ref=recv_ref.at[dst_slot],
                send_sem=send_sems.at[sem],
                recv_sem=recv_sems.at[dst_slot],
                device_id=(peer,),
                device_id_type=pl.DeviceIdType.MESH,
            )
            rdma.start()
            return rdma

        def wait_slot(slot):
            pltpu.make_async_remote_copy(
                src_ref=stage_ref.at[0],
                dst_ref=recv_ref.at[slot],
                send_sem=send_sems.at[0],
                recv_sem=recv_sems.at[slot],
                device_id=(left,),
                device_id_type=pl.DeviceIdType.MESH,
            ).wait_recv()

        d = pchunk((my + 2) % N_DEV)
        stage_ref[0, :, :] = d[:h_per]
        stage_ref[1, :, :] = d[h_per:]

        pl.semaphore_wait(barrier_sem, 2)
        rdmas = [send(0, 1, left, 0), send(1, 0, right, 1)]

        d = pchunk(right)
        stage_ref[2, :, :] = d[:h_per]
        stage_ref[4, :, :] = d[h_per:]
        rdmas.append(send(2, 2, right, 2))

        d = pchunk(left)
        stage_ref[5, :, :] = d[:h_per]
        stage_ref[3, :, :] = d[h_per:]
        rdmas.append(send(3, 3, left, 3))

        own_ref[...] = pchunk(my).reshape(2, h_per, N)

        wait_slot(0)
        stage_ref[4, :, :] = (
            stage_ref[4].astype(jnp.float32) + recv_ref[0].astype(jnp.float32)
        ).astype(jnp.bfloat16)
        rdmas.append(send(4, 5, right, 4))

        wait_slot(1)
        stage_ref[5, :, :] = (
            stage_ref[5].astype(jnp.float32) + recv_ref[1].astype(jnp.float32)
        ).astype(jnp.bfloat16)
        rdmas.append(send(5, 4, left, 5))

        wait_slot(2)
        wait_slot(4)
        out_ref[:h_per, :] = (
            own_ref[0].astype(jnp.float32)
            + recv_ref[2].astype(jnp.float32)
            + recv_ref[4].astype(jnp.float32)
        ).astype(jnp.bfloat16)

        wait_slot(3)
        wait_slot(5)
        out_ref[h_per:, :] = (
            own_ref[1].astype(jnp.float32)
            + recv_ref[3].astype(jnp.float32)
            + recv_ref[5].astype(jnp.float32)
        ).astype(jnp.bfloat16)

        for rdma in rdmas:
            rdma.wait_send()

    return pl.pallas_call(
        body,
        out_shape=jax.ShapeDtypeStruct((m_per, N), jnp.bfloat16),
        in_specs=[
            pl.BlockSpec(memory_space=pltpu.VMEM),
            pl.BlockSpec(memory_space=pltpu.VMEM),
        ],
        out_specs=pl.BlockSpec(memory_space=pltpu.VMEM),
        scratch_shapes=[
            pltpu.VMEM((6, h_per, N), jnp.bfloat16),
            pltpu.VMEM((2, h_per, N), jnp.bfloat16),
            pltpu.VMEM((6, h_per, N), jnp.bfloat16),
            pltpu.SemaphoreType.DMA((6,)),
            pltpu.SemaphoreType.DMA((6,)),
        ],
        compiler_params=pltpu.CompilerParams(collective_id=0),
    )(A, B)
